# baseline (speedup 1.0000x reference)
import functools

import jax
import jax.numpy as jnp
from jax import lax
from jax.experimental import pallas as pl
from jax.experimental.pallas import tpu as pltpu
from jax.experimental.pallas import tpu_sc as plsc

_pallas_call = pl.pallas_call
_pl_kernel = pl.kernel

_C = 80
_NW = 32
_NSUB = 16


def _expand_body(comp_ref, bases_ref, h_ref, v_ref):
    c = comp_ref[0, 0]
    w = jnp.sum(c[:, None, None] * bases_ref[...], axis=0)
    v_ref[0] = jnp.dot(h_ref[...], w.astype(jnp.bfloat16),
                       preferred_element_type=jnp.float32)


def _expand(h, bases_ext, comp_ext):
    rp, bp = comp_ext.shape
    n, d = h.shape
    return _pallas_call(
        _expand_body,
        grid=(rp,),
        in_specs=[
            pl.BlockSpec((1, 1, bp), lambda r: (r, 0, 0)),
            pl.BlockSpec((bp, d, d), lambda r: (0, 0, 0)),
            pl.BlockSpec((n, d), lambda r: (0, 0)),
        ],
        out_specs=pl.BlockSpec((1, n, d), lambda r: (r, 0, 0)),
        out_shape=jax.ShapeDtypeStruct((rp, n, d), jnp.float32),
    )(comp_ext.reshape(rp, 1, bp), bases_ext, h)


def _combine_expand_body(comp_ref, bases_ref, agg_ref, vloop_ref, bias_ref,
                         v_ref, h_scr):
    r = pl.program_id(0)

    @pl.when(r == 0)
    def _():
        h_scr[...] = jnp.maximum(
            agg_ref[0] + agg_ref[1] + vloop_ref[0] + bias_ref[...],
            0.0).astype(jnp.bfloat16)

    c = comp_ref[0, 0]
    w = jnp.sum(c[:, None, None] * bases_ref[...], axis=0)
    v_ref[0] = jnp.dot(h_scr[...], w.astype(jnp.bfloat16),
                       preferred_element_type=jnp.float32)


def _combine_expand(agg, v_prev, bias, bases_ext, comp_ext):
    rp_prev, n, d = v_prev.shape
    rp, bp = comp_ext.shape
    return _pallas_call(
        _combine_expand_body,
        grid=(rp,),
        in_specs=[
            pl.BlockSpec((1, 1, bp), lambda r: (r, 0, 0)),
            pl.BlockSpec((bp, d, d), lambda r: (0, 0, 0)),
            pl.BlockSpec((2, n, d), lambda r: (0, 0, 0)),
            pl.BlockSpec((1, n, d), lambda r: (rp_prev - 1, 0, 0)),
            pl.BlockSpec((1, d), lambda r: (0, 0)),
        ],
        out_specs=pl.BlockSpec((1, n, d), lambda r: (r, 0, 0)),
        out_shape=jax.ShapeDtypeStruct((rp, n, d), jnp.float32),
        scratch_shapes=[pltpu.VMEM((n, d), jnp.bfloat16)],
    )(comp_ext.reshape(rp, 1, bp), bases_ext, agg, v_prev,
      bias.reshape(1, d))


def _combine_body(agg_ref, vloop_ref, bias_ref, out_ref, *, relu):
    x = agg_ref[0] + agg_ref[1] + vloop_ref[0] + bias_ref[...]
    out_ref[...] = jnp.maximum(x, 0.0) if relu else x


def _combine(agg, v_ext, bias, relu):
    rp, n, d = v_ext.shape
    return _pallas_call(
        functools.partial(_combine_body, relu=relu),
        grid=(1,),
        in_specs=[
            pl.BlockSpec((2, n, d), lambda i: (0, 0, 0)),
            pl.BlockSpec((1, n, d), lambda i: (rp - 1, 0, 0)),
            pl.BlockSpec((1, d), lambda i: (0, 0)),
        ],
        out_specs=pl.BlockSpec((n, d), lambda i: (0, 0)),
        out_shape=jax.ShapeDtypeStruct((n, d), jnp.float32),
    )(agg, v_ext, bias.reshape(1, d))


def _edge_pass(ed_flat, v_flat, zeros_nd, n):
    e3 = ed_flat.shape[0]
    e = e3 // 3
    npad, d = zeros_nd.shape
    ept = e // _NW
    nchunks = ept // _C
    nch_total = e // _C
    ns = 4
    ngrp = nchunks // ns
    edc = 3 * _C
    rpt = npad // _NSUB
    mesh = plsc.VectorSubcoreMesh(core_axis_name="c", subcore_axis_name="s")

    @functools.partial(
        _pl_kernel,
        out_type=jax.ShapeDtypeStruct((2 * npad, d), jnp.float32),
        mesh=mesh,
        scratch_types=(
            [pltpu.VMEM((edc,), jnp.int32) for _ in range(ns)] +
            [pltpu.VMEM((_C,), jnp.int32) for _ in range(ns)] +
            [pltpu.VMEM((_C,), jnp.int32) for _ in range(ns)] +
            [pltpu.VMEM((_C, d), jnp.float32) for _ in range(ns)] +
            [pltpu.VMEM_SHARED((npad, d), jnp.float32)] +
            [pltpu.SemaphoreType.DMA for _ in range(3 * ns)]
        ),
    )
    def body(ed_hbm, v_hbm, z_hbm, out_hbm, *scr):
        cid = lax.axis_index("c")
        sid = lax.axis_index("s")
        wid = sid * 2 + cid
        edb = scr[0:ns]
        dstb = scr[ns:2 * ns]
        gidxb = scr[2 * ns:3 * ns]
        rowsb = scr[3 * ns:4 * ns]
        agg_s = scr[4 * ns]
        sem_i = scr[4 * ns + 1:4 * ns + 1 + ns]
        sem_g = scr[4 * ns + 1 + ns:4 * ns + 1 + 2 * ns]
        sem_s = scr[4 * ns + 1 + 2 * ns:4 * ns + 1 + 3 * ns]

        pltpu.sync_copy(z_hbm.at[pl.ds(sid * rpt, rpt)],
                        agg_s.at[pl.ds(sid * rpt, rpt)])
        plsc.subcore_barrier()

        cbase = wid * nchunks

        def idx_off(i):
            g = jnp.minimum(cbase + i, nch_total - 1)
            return pl.multiple_of(g * edc, 8)

        def issue_idx(i, s):
            pltpu.async_copy(ed_hbm.at[pl.ds(idx_off(i), edc)],
                             edb[s], sem_i[s])

        def wait_idx(s):
            pltpu.make_async_copy(ed_hbm.at[pl.ds(0, edc)],
                                  edb[s], sem_i[s]).wait()

        def prep(s):
            for j in range(_C // 16):
                sl = pl.ds(j * 16, 16)
                gidxb[s][sl] = edb[s][pl.ds(_C + j * 16, 16)] * n + edb[s][sl]
                dstb[s][sl] = edb[s][pl.ds(2 * _C + j * 16, 16)]

        def issue_gather(s):
            pltpu.async_copy(v_hbm.at[gidxb[s]], rowsb[s], sem_g[s])

        def wait_gather(s):
            pltpu.make_async_copy(v_hbm.at[gidxb[s]],
                                  rowsb[s], sem_g[s]).wait()

        def issue_scatter(s):
            return pltpu.async_copy(rowsb[s], agg_s.at[dstb[s]],
                                    sem_s[s], add=True)

        for s in range(ns):
            issue_idx(s, s)
        for s in range(ns):
            wait_idx(s)
            prep(s)
            issue_gather(s)

        def grp(k, carry):
            i = ns * k
            scs = []
            for s in range(ns):
                wait_gather(s)
                scs.append(issue_scatter(s))
                issue_idx(i + ns + s, s)
            for s in range(ns):
                wait_idx(s)
                scs[s].wait()
                prep(s)
                issue_gather(s)
            return carry

        lax.fori_loop(0, ngrp, grp, 0)

        ntail = nchunks - ns * ngrp
        last = []
        for s in range(ns):
            wait_gather(s)
            if s < ntail:
                last.append(issue_scatter(s))
        for h in last:
            h.wait()

        plsc.subcore_barrier()
        pltpu.sync_copy(agg_s.at[pl.ds(sid * rpt, rpt)],
                        out_hbm.at[pl.ds(cid * npad + sid * rpt, rpt)])

    return body(ed_flat, v_flat, zeros_nd)


def _ext_weights(bases, comp, loop_w):
    b = bases.shape[0]
    bases_ext = jnp.concatenate([bases, loop_w[None]], axis=0)
    loop_row = jnp.zeros((1, b + 1), comp.dtype).at[0, b].set(1.0)
    comp_ext = jnp.concatenate(
        [jnp.pad(comp, ((0, 0), (0, 1))), loop_row], axis=0)
    return bases_ext, comp_ext


def kernel(edge_index, etypes, embed, bases1, comp1, loop_w1, bias1,
           bases2, comp2, loop_w2, bias2):
    src = edge_index[0].astype(jnp.int32)
    dst = edge_index[1].astype(jnp.int32)
    ety = etypes.astype(jnp.int32)
    n, d = embed.shape
    npad = ((n + _NSUB * 8 - 1) // (_NSUB * 8)) * (_NSUB * 8)
    zeros_nd = jnp.zeros((npad, d), jnp.float32)
    ed_flat = jnp.stack(
        [src.reshape(-1, _C), ety.reshape(-1, _C), dst.reshape(-1, _C)],
        axis=1).reshape(-1)
    be1, ce1 = _ext_weights(bases1, comp1, loop_w1)
    be2, ce2 = _ext_weights(bases2, comp2, loop_w2)
    rp = ce1.shape[0]

    v1 = _expand(embed.astype(jnp.bfloat16), be1, ce1)
    agg1 = _edge_pass(ed_flat, v1.reshape(rp * n, d), zeros_nd, n)
    v2 = _combine_expand(agg1.reshape(2, npad, d)[:, :n], v1, bias1,
                         be2, ce2)
    agg2 = _edge_pass(ed_flat, v2.reshape(rp * n, d), zeros_nd, n)
    return _combine(agg2.reshape(2, npad, d)[:, :n], v2, bias2, False)


# speedup vs baseline: 1.2054x; 1.2054x over previous
import functools

import jax
import jax.numpy as jnp
from jax import lax
from jax.experimental import pallas as pl
from jax.experimental.pallas import tpu as pltpu
from jax.experimental.pallas import tpu_sc as plsc

_pallas_call = pl.pallas_call
_pl_kernel = pl.kernel

_C = 80
_NW = 32
_NSUB = 16


def _expand_body(comp_ref, bases_ref, h_ref, v_ref):
    c = comp_ref[0, 0]
    w = jnp.sum(c[:, None, None] * bases_ref[...], axis=0)
    v_ref[...] = jnp.dot(h_ref[...], w, preferred_element_type=jnp.float32)


def _expand(h, bases_ext, comp_ext):
    rp, bp = comp_ext.shape
    n, d = h.shape
    return _pallas_call(
        _expand_body,
        grid=(rp,),
        in_specs=[
            pl.BlockSpec((1, 1, bp), lambda r: (r, 0, 0)),
            pl.BlockSpec((bp, d, d), lambda r: (0, 0, 0)),
            pl.BlockSpec((n, d), lambda r: (0, 0)),
        ],
        out_specs=pl.BlockSpec((n, d), lambda r: (r, 0)),
        out_shape=jax.ShapeDtypeStruct((rp * n, d), jnp.float32),
    )(comp_ext.reshape(rp, 1, bp), bases_ext, h)


def _combine_body(agg_ref, vloop_ref, bias_ref, out_ref, *, relu, n, npad):
    x = (agg_ref[pl.ds(0, n)] + agg_ref[pl.ds(npad, n)] + vloop_ref[...]
         + bias_ref[...])
    out_ref[...] = jnp.maximum(x, 0.0) if relu else x


def _combine(agg, v_flat, bias, n, relu):
    npad2, d = agg.shape
    npad = npad2 // 2
    rp = v_flat.shape[0] // n
    return _pallas_call(
        functools.partial(_combine_body, relu=relu, n=n, npad=npad),
        grid=(1,),
        in_specs=[
            pl.BlockSpec((npad2, d), lambda i: (0, 0)),
            pl.BlockSpec((n, d), lambda i: (rp - 1, 0)),
            pl.BlockSpec((1, d), lambda i: (0, 0)),
        ],
        out_specs=pl.BlockSpec((n, d), lambda i: (0, 0)),
        out_shape=jax.ShapeDtypeStruct((n, d), jnp.float32),
    )(agg, v_flat, bias.reshape(1, d))


def _edge_pass(ei, et, v_flat, zeros_nd, n):
    e = ei.shape[0] // 2
    npad, d = zeros_nd.shape
    ept = e // _NW
    nchunks = ept // _C
    nch_total = e // _C
    ns = 4
    ngrp = nchunks // ns
    rpt = npad // _NSUB
    mesh = plsc.VectorSubcoreMesh(core_axis_name="c", subcore_axis_name="s")

    @functools.partial(
        _pl_kernel,
        out_type=jax.ShapeDtypeStruct((2 * npad, d), jnp.float32),
        mesh=mesh,
        scratch_types=(
            [pltpu.VMEM((_C,), jnp.int32) for _ in range(ns)] +
            [pltpu.VMEM((_C,), jnp.int32) for _ in range(ns)] +
            [pltpu.VMEM((_C,), jnp.int32) for _ in range(ns)] +
            [pltpu.VMEM((_C,), jnp.int32) for _ in range(ns)] +
            [pltpu.VMEM((_C, d), jnp.float32) for _ in range(ns)] +
            [pltpu.VMEM_SHARED((npad, d), jnp.float32)] +
            [pltpu.SemaphoreType.DMA for _ in range(5 * ns)]
        ),
    )
    def body(ei_hbm, et_hbm, v_hbm, z_hbm, out_hbm, *scr):
        cid = lax.axis_index("c")
        sid = lax.axis_index("s")
        wid = sid * 2 + cid
        srcb = scr[0:ns]
        etyb = scr[ns:2 * ns]
        dstb = scr[2 * ns:3 * ns]
        gidxb = scr[3 * ns:4 * ns]
        rowsb = scr[4 * ns:5 * ns]
        agg_s = scr[5 * ns]
        sem_i = scr[5 * ns + 1:5 * ns + 1 + ns]
        sem_e = scr[5 * ns + 1 + ns:5 * ns + 1 + 2 * ns]
        sem_d = scr[5 * ns + 1 + 2 * ns:5 * ns + 1 + 3 * ns]
        sem_g = scr[5 * ns + 1 + 3 * ns:5 * ns + 1 + 4 * ns]
        sem_s = scr[5 * ns + 1 + 4 * ns:5 * ns + 1 + 5 * ns]

        pltpu.sync_copy(z_hbm.at[pl.ds(sid * rpt, rpt)],
                        agg_s.at[pl.ds(sid * rpt, rpt)])
        plsc.subcore_barrier()

        cbase = wid * nchunks

        def chunk_off(i):
            g = jnp.minimum(cbase + i, nch_total - 1)
            return pl.multiple_of(g * _C, 8)

        def issue_se(i, s):
            off = chunk_off(i)
            pltpu.async_copy(ei_hbm.at[pl.ds(off, _C)], srcb[s], sem_i[s])
            pltpu.async_copy(et_hbm.at[pl.ds(off, _C)], etyb[s], sem_e[s])

        def wait_se(s):
            pltpu.make_async_copy(ei_hbm.at[pl.ds(0, _C)],
                                  srcb[s], sem_i[s]).wait()
            pltpu.make_async_copy(et_hbm.at[pl.ds(0, _C)],
                                  etyb[s], sem_e[s]).wait()

        def issue_d(i, s):
            off2 = pl.multiple_of(chunk_off(i) + e, 8)
            pltpu.async_copy(ei_hbm.at[pl.ds(off2, _C)], dstb[s], sem_d[s])

        def wait_d(s):
            pltpu.make_async_copy(ei_hbm.at[pl.ds(0, _C)],
                                  dstb[s], sem_d[s]).wait()

        def prep(s):
            for j in range(_C // 16):
                sl = pl.ds(j * 16, 16)
                gidxb[s][sl] = etyb[s][sl] * n + srcb[s][sl]

        def issue_gather(s):
            pltpu.async_copy(v_hbm.at[gidxb[s]], rowsb[s], sem_g[s])

        def wait_gather(s):
            pltpu.make_async_copy(v_hbm.at[gidxb[s]],
                                  rowsb[s], sem_g[s]).wait()

        def issue_scatter(s):
            return pltpu.async_copy(rowsb[s], agg_s.at[dstb[s]],
                                    sem_s[s], add=True)

        for s in range(ns):
            issue_se(s, s)
            issue_d(s, s)
        for s in range(ns):
            wait_se(s)
            prep(s)
            issue_gather(s)

        def grp(k, carry):
            i = ns * k
            scs = []
            for s in range(ns):
                wait_gather(s)
                wait_d(s)
                scs.append(issue_scatter(s))
                issue_se(i + ns + s, s)
            for s in range(ns):
                wait_se(s)
                scs[s].wait()
                issue_d(i + ns + s, s)
                prep(s)
                issue_gather(s)
            return carry

        lax.fori_loop(0, ngrp, grp, 0)

        ntail = nchunks - ns * ngrp
        last = []
        for s in range(ns):
            wait_gather(s)
            wait_d(s)
            if s < ntail:
                last.append(issue_scatter(s))
        for h in last:
            h.wait()

        plsc.subcore_barrier()
        pltpu.sync_copy(agg_s.at[pl.ds(sid * rpt, rpt)],
                        out_hbm.at[pl.ds(cid * npad + sid * rpt, rpt)])

    return body(ei, et, v_flat, zeros_nd)


def _ext_weights(bases, comp, loop_w):
    b = bases.shape[0]
    bases_ext = jnp.concatenate([bases, loop_w[None]], axis=0)
    loop_row = jnp.zeros((1, b + 1), comp.dtype).at[0, b].set(1.0)
    comp_ext = jnp.concatenate(
        [jnp.pad(comp, ((0, 0), (0, 1))), loop_row], axis=0)
    return bases_ext, comp_ext


def kernel(edge_index, etypes, embed, bases1, comp1, loop_w1, bias1,
           bases2, comp2, loop_w2, bias2):
    ei = edge_index.astype(jnp.int32).reshape(-1)
    et = etypes.astype(jnp.int32)
    n, d = embed.shape
    npad = ((n + _NSUB * 8 - 1) // (_NSUB * 8)) * (_NSUB * 8)
    zeros_nd = jnp.zeros((npad, d), jnp.float32)
    be1, ce1 = _ext_weights(bases1, comp1, loop_w1)
    be2, ce2 = _ext_weights(bases2, comp2, loop_w2)

    v1 = _expand(embed, be1, ce1)
    agg1 = _edge_pass(ei, et, v1, zeros_nd, n)
    h1 = _combine(agg1, v1, bias1, n, True)
    v2 = _expand(h1, be2, ce2)
    agg2 = _edge_pass(ei, et, v2, zeros_nd, n)
    return _combine(agg2, v2, bias2, n, False)
